# baseline (speedup 1.0000x reference)
import functools

import jax
import jax.numpy as jnp
from jax import lax
from jax.experimental import pallas as pl
from jax.experimental.pallas import tpu as pltpu
from jax.experimental.pallas import tpu_sc as plsc

_L = 16
_PR = 6144
_NW = 32


def _make_kwinners(n_rt, C, RT):
  assert n_rt % (_NW * RT) == 0 and (n_rt // (_NW * RT)) % 2 == 0
  rt_w = n_rt // _NW
  n_chunks = rt_w // RT
  n_grp = RT // 2
  chunk = RT * C * 8
  n_step = C // _L
  mesh = plsc.VectorSubcoreMesh(core_axis_name="c", subcore_axis_name="s")

  @functools.partial(
      pl.kernel,
      mesh=mesh,
      out_type=jax.ShapeDtypeStruct((n_rt * C * 8,), jnp.float32),
      compiler_params=pltpu.CompilerParams(needs_layout_passes=False),
      scratch_types=[
          pltpu.VMEM((chunk,), jnp.float32),
          pltpu.VMEM((chunk,), jnp.float32),
          pltpu.VMEM((chunk,), jnp.float32),
          pltpu.VMEM((chunk,), jnp.float32),
          pltpu.SemaphoreType.DMA,
          pltpu.SemaphoreType.DMA,
          pltpu.SemaphoreType.DMA,
          pltpu.SemaphoreType.DMA,
      ],
  )
  def kw(x_hbm, o_hbm, ibuf_a, ibuf_b, obuf_a, obuf_b,
         rsem_a, rsem_b, wsem_a, wsem_b):
    wid = lax.axis_index("s") * 2 + lax.axis_index("c")
    base = wid * rt_w * _PR
    lane = lax.iota(jnp.int32, _L)
    pixoff = (lane >> 3) * _PR + (lane & 7) * 128
    neg_inf = jnp.full((_L,), -jnp.inf, jnp.float32)
    zero_i = jnp.zeros((_L,), jnp.int32)
    zero_f = jnp.zeros((_L,), jnp.float32)
    ibufs = (ibuf_a, ibuf_b)
    obufs = (obuf_a, obuf_b)
    rsems = (rsem_a, rsem_b)
    wsems = (wsem_a, wsem_b)

    def rd(ci):
      return x_hbm.at[pl.ds(base + ci * chunk, chunk)]

    def wr(ci):
      return o_hbm.at[pl.ds(base + ci * chunk, chunk)]

    pltpu.async_copy(rd(0), ibuf_a, rsem_a)
    pltpu.async_copy(rd(1), ibuf_b, rsem_b)

    def chunk_body(g, _):
      for p in range(2):
        ci = 2 * g + p
        pltpu.make_async_copy(rd(ci), ibufs[p], rsems[p]).wait()

        @pl.when(g > 0)
        def _():
          pltpu.make_async_copy(obufs[p], wr(ci - 2), wsems[p]).wait()

        ib = ibufs[p]
        ob = obufs[p]

        for gi in range(n_grp):
          gbase = gi * 2 * _PR
          base_vec = pixoff + gbase

          def step(s, carry):
            ms, is_ = list(carry[:4]), list(carry[4:])
            off = (s >> 3) * 1024 + (s & 7) * _L
            idx0 = base_vec + off
            zbase = gbase + s * 256
            for j in range(_L):
              idx = idx0 + j if j else idx0
              v = plsc.load_gather(ib, [idx])
              a = j & 3
              gt = v > ms[a]
              ms[a] = jnp.where(gt, v, ms[a])
              is_[a] = jnp.where(gt, idx, is_[a])
              ob[pl.ds(zbase + j * _L, _L)] = zero_f
            return tuple(ms) + tuple(is_)

          acc = lax.fori_loop(
              0, n_step, step,
              (neg_inf,) * 4 + (zero_i,) * 4)
          ms, is_ = acc[:4], acc[4:]

          def mrg(ma, ia, mb, ib_):
            t = (ma > mb) | ((ma == mb) & (ia < ib_))
            return jnp.where(t, ma, mb), jnp.where(t, ia, ib_)

          m0, i0 = mrg(ms[0], is_[0], ms[1], is_[1])
          m1, i1 = mrg(ms[2], is_[2], ms[3], is_[3])
          mw, iw = mrg(m0, i0, m1, i1)
          plsc.store_scatter(ob, [iw], mw)

        pltpu.async_copy(ob, wr(ci), wsems[p])

        @pl.when(ci + 2 < n_chunks)
        def _():
          pltpu.async_copy(rd(ci + 2), ibufs[p], rsems[p])

      return 0

    lax.fori_loop(0, n_chunks // 2, chunk_body, 0)

    for p in range(2):
      pltpu.make_async_copy(obufs[p], wr(n_chunks - 2 + p), wsems[p]).wait()

  return kw


def kernel(x, k):
  B, C, H, W = x.shape
  n_rt = B * H * W // 8
  n_ct = C // 128
  xt = jnp.transpose(x, (0, 2, 3, 1))
  x4 = jnp.transpose(xt.reshape(n_rt, 8, n_ct, 128), (0, 2, 1, 3))
  xf = x4.reshape(n_rt * C * 8)
  of = _make_kwinners(n_rt, C, 4)(xf)
  o4 = of.reshape(n_rt, n_ct, 8, 128)
  ot = jnp.transpose(o4, (0, 2, 1, 3)).reshape(B, H, W, C)
  return jnp.transpose(ot, (0, 3, 1, 2))


# speedup vs baseline: 3.2635x; 3.2635x over previous
import functools

import jax
import jax.numpy as jnp
from jax import lax
from jax.experimental import pallas as pl
from jax.experimental.pallas import tpu as pltpu
from jax.experimental.pallas import tpu_sc as plsc

_L = 16
_PR = 6144
_NW = 32


def _make_kwinners(n_rt, C, RT):
  assert n_rt % (_NW * RT) == 0 and (n_rt // (_NW * RT)) % 2 == 0
  rt_w = n_rt // _NW
  n_chunks = rt_w // RT
  pix = RT * 8
  chunk = RT * _PR
  n_k = C // _L
  mesh = plsc.VectorSubcoreMesh(core_axis_name="c", subcore_axis_name="s")

  @functools.partial(
      pl.kernel,
      mesh=mesh,
      out_type=jax.ShapeDtypeStruct((n_rt * _PR,), jnp.float32),
      compiler_params=pltpu.CompilerParams(needs_layout_passes=False),
      scratch_types=[
          pltpu.VMEM((chunk,), jnp.float32),
          pltpu.VMEM((chunk,), jnp.float32),
          pltpu.VMEM((chunk,), jnp.float32),
          pltpu.VMEM((chunk,), jnp.float32),
          pltpu.SMEM((pix,), jnp.int32),
          pltpu.SMEM((pix,), jnp.int32),
          pltpu.SemaphoreType.DMA,
          pltpu.SemaphoreType.DMA,
          pltpu.SemaphoreType.DMA,
          pltpu.SemaphoreType.DMA,
      ],
  )
  def kw(x_hbm, o_hbm, ibuf_a, ibuf_b, obuf_a, obuf_b, wsl_a, wsl_b,
         rsem_a, rsem_b, wsem_a, wsem_b):
    wid = lax.axis_index("s") * 2 + lax.axis_index("c")
    base = wid * rt_w * _PR
    lane = lax.iota(jnp.int32, _L)
    neg_inf = jnp.full((_L,), -jnp.inf, jnp.float32)
    zero_f = jnp.zeros((_L,), jnp.float32)
    big_i = jnp.full((_L,), C, jnp.int32)
    idx_init = tuple(a * _L + lane for a in range(4))
    ibufs = (ibuf_a, ibuf_b)
    obufs = (obuf_a, obuf_b)
    wsls = (wsl_a, wsl_b)
    rsems = (rsem_a, rsem_b)
    wsems = (wsem_a, wsem_b)

    def rd(ci):
      return x_hbm.at[pl.ds(base + ci * chunk, chunk)]

    def wr(ci):
      return o_hbm.at[pl.ds(base + ci * chunk, chunk)]

    for ob in obufs:
      def zinit(j, _):
        ob[pl.ds(j * _L, _L)] = zero_f
        return 0

      lax.fori_loop(0, chunk // _L, zinit, 0)

    pltpu.async_copy(rd(0), ibuf_a, rsem_a)
    pltpu.async_copy(rd(1), ibuf_b, rsem_b)

    def chunk_body(g, _):
      for p in range(2):
        ci = 2 * g + p
        pltpu.make_async_copy(rd(ci), ibufs[p], rsems[p]).wait()
        ib = ibufs[p]
        ob = obufs[p]
        wsl = wsls[p]

        @pl.when(g > 0)
        def _():
          pltpu.make_async_copy(obufs[p], wr(ci - 2), wsems[p]).wait()

          def rz(q, _):
            ob[pl.ds(wsl[q], _L)] = zero_f
            return 0

          lax.fori_loop(0, pix, rz, 0)

        def pix_body(q, _):
          pb = (q >> 3) * _PR + (q & 7) * 128
          ms = [neg_inf] * 4
          is_ = list(idx_init)
          iv = list(idx_init)
          for kk in range(n_k):
            a = kk & 3
            v = ib[pl.ds(pb + (kk >> 3) * 1024 + (kk & 7) * _L, _L)]
            gt = v > ms[a]
            ms[a] = jnp.where(gt, v, ms[a])
            is_[a] = jnp.where(gt, iv[a], is_[a])
            if kk + 4 < n_k:
              iv[a] = iv[a] + 4 * _L
          def mrg(ma, ia, mb, ib2):
            t = (ma > mb) | ((ma == mb) & (ia < ib2))
            return jnp.where(t, ma, mb), jnp.where(t, ia, ib2)

          m0, i0 = mrg(ms[0], is_[0], ms[1], is_[1])
          m1, i1 = mrg(ms[2], is_[2], ms[3], is_[3])
          mv, ixv = mrg(m0, i0, m1, i1)
          mx = jnp.max(mv)
          wi = jnp.min(jnp.where(mv == mx, ixv, big_i))
          slot = pb + (wi >> 7) * 1024 + ((wi & 127) >> 4) * _L
          ob[pl.ds(slot, _L)] = jnp.where(
              lane == (wi & 15), mx, jnp.float32(0))
          wsl[q] = slot
          return 0

        lax.fori_loop(0, pix, pix_body, 0)
        pltpu.async_copy(ob, wr(ci), wsems[p])

        @pl.when(ci + 2 < n_chunks)
        def _():
          pltpu.async_copy(rd(ci + 2), ibufs[p], rsems[p])

      return 0

    lax.fori_loop(0, n_chunks // 2, chunk_body, 0)

    for p in range(2):
      pltpu.make_async_copy(obufs[p], wr(n_chunks - 2 + p), wsems[p]).wait()

  return kw


def kernel(x, k):
  B, C, H, W = x.shape
  n_rt = B * H * W // 8
  n_ct = C // 128
  xt = jnp.transpose(x, (0, 2, 3, 1))
  x4 = jnp.transpose(xt.reshape(n_rt, 8, n_ct, 128), (0, 2, 1, 3))
  xf = x4.reshape(n_rt * C * 8)
  of = _make_kwinners(n_rt, C, 4)(xf)
  o4 = of.reshape(n_rt, n_ct, 8, 128)
  ot = jnp.transpose(o4, (0, 2, 1, 3)).reshape(B, H, W, C)
  return jnp.transpose(ot, (0, 3, 1, 2))
